# baseline (speedup 1.0000x reference)
import functools

import jax
import jax.numpy as jnp
from jax import lax
from jax.experimental import pallas as pl
from jax.experimental.pallas import tpu as pltpu
from jax.experimental.pallas import tpu_sc as plsc

_INPUT_DIM = 100000
_LATENT_DIM = 64
_BATCH = 1024

_LANES = 16


def _sc_gather_sigmoid(x, encoder_weight):
    info = plsc.get_sparse_core_info()
    nc, ns = info.num_cores, info.num_subcores
    nw = nc * ns
    b_per_w = _BATCH // nw
    mesh = plsc.VectorSubcoreMesh(core_axis_name="c", subcore_axis_name="s")

    @functools.partial(
        pl.kernel,
        mesh=mesh,
        compiler_params=pltpu.CompilerParams(use_tc_tiling_on_sc=False),
        out_type=jax.ShapeDtypeStruct((_BATCH, _LATENT_DIM), jnp.float32),
        scratch_types=[
            pltpu.VMEM((b_per_w,), jnp.int32),
            pltpu.VMEM((b_per_w, _LATENT_DIM), jnp.float32),
            pltpu.SemaphoreType.DMA,
        ],
    )
    def body(x_hbm, table_hbm, out_hbm, idx_v, rows_v, sem):
        wid = lax.axis_index("s") * nc + lax.axis_index("c")
        base = wid * b_per_w
        pltpu.sync_copy(x_hbm.at[pl.ds(base, b_per_w)], idx_v)
        pltpu.async_copy(table_hbm.at[idx_v], rows_v, sem).wait()
        for i in range(b_per_w):
            for j in range(_LATENT_DIM // _LANES):
                v = rows_v[i, pl.ds(j * _LANES, _LANES)]
                rows_v[i, pl.ds(j * _LANES, _LANES)] = 1.0 / (1.0 + jnp.exp(-v))
        pltpu.sync_copy(rows_v, out_hbm.at[pl.ds(base, b_per_w)])

    return body(x, encoder_weight)


_TILE_M = 32


def _mm_body(h_ref, d_ref, o_ref):
    acc = jnp.dot(h_ref[...], d_ref[...], preferred_element_type=jnp.float32)
    o_ref[...] = 1.0 / (1.0 + jnp.exp(-acc))


def _tc_decode(h, decoder_weight):
    return pl.pallas_call(
        _mm_body,
        grid=(_BATCH // _TILE_M,),
        in_specs=[
            pl.BlockSpec((_TILE_M, _LATENT_DIM), lambda i: (i, 0)),
            pl.BlockSpec((_LATENT_DIM, _INPUT_DIM), lambda i: (0, 0)),
        ],
        out_specs=pl.BlockSpec((_TILE_M, _INPUT_DIM), lambda i: (i, 0)),
        out_shape=jax.ShapeDtypeStruct((_BATCH, _INPUT_DIM), jnp.float32),
    )(h, decoder_weight)


def kernel(x, encoder_weight, decoder_weight):
    h = _sc_gather_sigmoid(x.astype(jnp.int32), encoder_weight)
    return _tc_decode(h, decoder_weight)


# speedup vs baseline: 2.4812x; 2.4812x over previous
import functools

import jax
import jax.numpy as jnp
from jax import lax
from jax.experimental import pallas as pl
from jax.experimental.pallas import tpu as pltpu
from jax.experimental.pallas import tpu_sc as plsc

_INPUT_DIM = 100000
_LATENT_DIM = 64
_BATCH = 1024

_LANES = 16


def _sc_gather_sigmoid(x, encoder_weight):
    info = plsc.get_sparse_core_info()
    nc, ns = info.num_cores, info.num_subcores
    nw = nc * ns
    b_per_w = _BATCH // nw
    mesh = plsc.VectorSubcoreMesh(core_axis_name="c", subcore_axis_name="s")

    @functools.partial(
        pl.kernel,
        mesh=mesh,
        compiler_params=pltpu.CompilerParams(use_tc_tiling_on_sc=False),
        out_type=jax.ShapeDtypeStruct((_BATCH, _LATENT_DIM), jnp.float32),
        scratch_types=[
            pltpu.VMEM((b_per_w,), jnp.int32),
            pltpu.VMEM((b_per_w, _LATENT_DIM), jnp.float32),
            pltpu.SemaphoreType.DMA,
        ],
    )
    def body(x_hbm, table_hbm, out_hbm, idx_v, rows_v, sem):
        wid = lax.axis_index("s") * nc + lax.axis_index("c")
        base = wid * b_per_w
        pltpu.sync_copy(x_hbm.at[pl.ds(base, b_per_w)], idx_v)
        pltpu.async_copy(table_hbm.at[idx_v], rows_v, sem).wait()
        for i in range(b_per_w):
            for j in range(_LATENT_DIM // _LANES):
                v = rows_v[i, pl.ds(j * _LANES, _LANES)]
                rows_v[i, pl.ds(j * _LANES, _LANES)] = 1.0 / (1.0 + jnp.exp(-v))
        pltpu.sync_copy(rows_v, out_hbm.at[pl.ds(base, b_per_w)])

    return body(x, encoder_weight)


_TILE_V = 2048


def _mm_body(h_ref, d_ref, o_ref):
    acc = jax.lax.dot_general(
        d_ref[...], h_ref[...],
        (((0,), (1,)), ((), ())),
        preferred_element_type=jnp.float32,
    )
    o_ref[...] = 1.0 / (1.0 + jnp.exp(-acc))


def _tc_decode(h, decoder_weight):
    ot = pl.pallas_call(
        _mm_body,
        grid=(pl.cdiv(_INPUT_DIM, _TILE_V),),
        in_specs=[
            pl.BlockSpec((_BATCH, _LATENT_DIM), lambda i: (0, 0)),
            pl.BlockSpec((_LATENT_DIM, _TILE_V), lambda i: (0, i)),
        ],
        out_specs=pl.BlockSpec((_TILE_V, _BATCH), lambda i: (i, 0)),
        out_shape=jax.ShapeDtypeStruct((_INPUT_DIM, _BATCH), jnp.float32),
    )(h, decoder_weight)
    return ot.T


def kernel(x, encoder_weight, decoder_weight):
    h = _sc_gather_sigmoid(x.astype(jnp.int32), encoder_weight)
    return _tc_decode(h, decoder_weight)


# speedup vs baseline: 2.5350x; 1.0217x over previous
import functools

import jax
import jax.numpy as jnp
from jax import lax
from jax.experimental import pallas as pl
from jax.experimental.pallas import tpu as pltpu
from jax.experimental.pallas import tpu_sc as plsc

_INPUT_DIM = 100000
_LATENT_DIM = 64
_BATCH = 1024

_LANES = 16


def _sc_gather_sigmoid(x, encoder_weight):
    info = plsc.get_sparse_core_info()
    nc, ns = info.num_cores, info.num_subcores
    nw = nc * ns
    b_per_w = _BATCH // nw
    mesh = plsc.VectorSubcoreMesh(core_axis_name="c", subcore_axis_name="s")

    @functools.partial(
        pl.kernel,
        mesh=mesh,
        compiler_params=pltpu.CompilerParams(use_tc_tiling_on_sc=False),
        out_type=jax.ShapeDtypeStruct((_BATCH, _LATENT_DIM), jnp.float32),
        scratch_types=[
            pltpu.VMEM((b_per_w,), jnp.int32),
            pltpu.VMEM((b_per_w, _LATENT_DIM), jnp.float32),
            pltpu.SemaphoreType.DMA,
        ],
    )
    def body(x_hbm, table_hbm, out_hbm, idx_v, rows_v, sem):
        wid = lax.axis_index("s") * nc + lax.axis_index("c")
        base = wid * b_per_w
        pltpu.sync_copy(x_hbm.at[pl.ds(base, b_per_w)], idx_v)
        pltpu.async_copy(table_hbm.at[idx_v], rows_v, sem).wait()
        pltpu.sync_copy(rows_v, out_hbm.at[pl.ds(base, b_per_w)])

    return body(x, encoder_weight)


_TILE_V = 2048


def _mm_body(h_ref, d_ref, o_ref):
    h = 1.0 / (1.0 + jnp.exp(-h_ref[...]))
    acc = jax.lax.dot_general(
        d_ref[...], h,
        (((0,), (1,)), ((), ())),
        preferred_element_type=jnp.float32,
    )
    o_ref[...] = 1.0 / (1.0 + jnp.exp(-acc))


def _tc_decode(h, decoder_weight):
    ot = pl.pallas_call(
        _mm_body,
        grid=(pl.cdiv(_INPUT_DIM, _TILE_V),),
        in_specs=[
            pl.BlockSpec((_BATCH, _LATENT_DIM), lambda i: (0, 0)),
            pl.BlockSpec((_LATENT_DIM, _TILE_V), lambda i: (0, i)),
        ],
        out_specs=pl.BlockSpec((_TILE_V, _BATCH), lambda i: (i, 0)),
        out_shape=jax.ShapeDtypeStruct((_INPUT_DIM, _BATCH), jnp.float32),
    )(h, decoder_weight)
    return ot.T


def kernel(x, encoder_weight, decoder_weight):
    h = _sc_gather_sigmoid(x.astype(jnp.int32), encoder_weight)
    return _tc_decode(h, decoder_weight)


# speedup vs baseline: 2.5697x; 1.0137x over previous
import functools

import jax
import jax.numpy as jnp
from jax import lax
from jax.experimental import pallas as pl
from jax.experimental.pallas import tpu as pltpu
from jax.experimental.pallas import tpu_sc as plsc

_INPUT_DIM = 100000
_LATENT_DIM = 64
_BATCH = 1024

_LANES = 16


def _sc_gather_sigmoid(x, encoder_weight):
    info = plsc.get_sparse_core_info()
    nc, ns = info.num_cores, info.num_subcores
    nw = nc * ns
    b_per_w = _BATCH // nw
    mesh = plsc.VectorSubcoreMesh(core_axis_name="c", subcore_axis_name="s")

    @functools.partial(
        pl.kernel,
        mesh=mesh,
        compiler_params=pltpu.CompilerParams(use_tc_tiling_on_sc=False),
        out_type=jax.ShapeDtypeStruct((_BATCH, _LATENT_DIM), jnp.float32),
        scratch_types=[
            pltpu.VMEM((b_per_w,), jnp.int32),
            pltpu.VMEM((b_per_w, _LATENT_DIM), jnp.float32),
            pltpu.SemaphoreType.DMA,
        ],
    )
    def body(x_hbm, table_hbm, out_hbm, idx_v, rows_v, sem):
        wid = lax.axis_index("s") * nc + lax.axis_index("c")
        base = wid * b_per_w
        pltpu.sync_copy(x_hbm.at[pl.ds(base, b_per_w)], idx_v)
        pltpu.async_copy(table_hbm.at[idx_v], rows_v, sem).wait()
        pltpu.sync_copy(rows_v, out_hbm.at[pl.ds(base, b_per_w)])

    return body(x, encoder_weight)


_TILE_V = 4096


def _mm_body(h_ref, d_ref, o_ref):
    h = 1.0 / (1.0 + jnp.exp(-h_ref[...]))
    acc = jax.lax.dot_general(
        d_ref[...], h,
        (((0,), (1,)), ((), ())),
        preferred_element_type=jnp.float32,
    )
    o_ref[...] = 1.0 / (1.0 + jnp.exp(-acc))


def _tc_decode(h, decoder_weight):
    ot = pl.pallas_call(
        _mm_body,
        grid=(pl.cdiv(_INPUT_DIM, _TILE_V),),
        in_specs=[
            pl.BlockSpec((_BATCH, _LATENT_DIM), lambda i: (0, 0)),
            pl.BlockSpec((_LATENT_DIM, _TILE_V), lambda i: (0, i)),
        ],
        out_specs=pl.BlockSpec((_TILE_V, _BATCH), lambda i: (i, 0)),
        out_shape=jax.ShapeDtypeStruct((_INPUT_DIM, _BATCH), jnp.float32),
    )(h, decoder_weight)
    return ot.T


def kernel(x, encoder_weight, decoder_weight):
    h = _sc_gather_sigmoid(x.astype(jnp.int32), encoder_weight)
    return _tc_decode(h, decoder_weight)


# speedup vs baseline: 2.5818x; 1.0047x over previous
import functools

import jax
import jax.numpy as jnp
from jax import lax
from jax.experimental import pallas as pl
from jax.experimental.pallas import tpu as pltpu
from jax.experimental.pallas import tpu_sc as plsc

_INPUT_DIM = 100000
_LATENT_DIM = 64
_BATCH = 1024

_LANES = 16


def _sc_gather_sigmoid(x, encoder_weight):
    info = plsc.get_sparse_core_info()
    nc, ns = info.num_cores, info.num_subcores
    nw = nc * ns
    b_per_w = _BATCH // nw
    mesh = plsc.VectorSubcoreMesh(core_axis_name="c", subcore_axis_name="s")

    @functools.partial(
        pl.kernel,
        mesh=mesh,
        compiler_params=pltpu.CompilerParams(use_tc_tiling_on_sc=False),
        out_type=jax.ShapeDtypeStruct((_BATCH, _LATENT_DIM), jnp.float32),
        scratch_types=[
            pltpu.VMEM((b_per_w,), jnp.int32),
            pltpu.VMEM((b_per_w, _LATENT_DIM), jnp.float32),
            pltpu.SemaphoreType.DMA,
        ],
    )
    def body(x_hbm, table_hbm, out_hbm, idx_v, rows_v, sem):
        wid = lax.axis_index("s") * nc + lax.axis_index("c")
        base = wid * b_per_w
        pltpu.sync_copy(x_hbm.at[pl.ds(base, b_per_w)], idx_v)
        pltpu.async_copy(table_hbm.at[idx_v], rows_v, sem).wait()
        pltpu.sync_copy(rows_v, out_hbm.at[pl.ds(base, b_per_w)])

    return body(x, encoder_weight)


_TILE_V = 6144


def _mm_body(h_ref, d_ref, o_ref):
    h = 1.0 / (1.0 + jnp.exp(-h_ref[...]))
    acc = jax.lax.dot_general(
        d_ref[...], h,
        (((0,), (1,)), ((), ())),
        preferred_element_type=jnp.float32,
    )
    o_ref[...] = 1.0 / (1.0 + jnp.exp(-acc))


def _tc_decode(h, decoder_weight):
    ot = pl.pallas_call(
        _mm_body,
        grid=(pl.cdiv(_INPUT_DIM, _TILE_V),),
        in_specs=[
            pl.BlockSpec((_BATCH, _LATENT_DIM), lambda i: (0, 0)),
            pl.BlockSpec((_LATENT_DIM, _TILE_V), lambda i: (0, i)),
        ],
        out_specs=pl.BlockSpec((_TILE_V, _BATCH), lambda i: (i, 0)),
        out_shape=jax.ShapeDtypeStruct((_INPUT_DIM, _BATCH), jnp.float32),
    )(h, decoder_weight)
    return ot.T


def kernel(x, encoder_weight, decoder_weight):
    h = _sc_gather_sigmoid(x.astype(jnp.int32), encoder_weight)
    return _tc_decode(h, decoder_weight)


# speedup vs baseline: 3.2655x; 1.2648x over previous
import functools

import jax
import jax.numpy as jnp
from jax import lax
from jax.experimental import pallas as pl
from jax.experimental.pallas import tpu as pltpu
from jax.experimental.pallas import tpu_sc as plsc

_INPUT_DIM = 100000
_LATENT_DIM = 64
_BATCH = 1024

_LANES = 16


_SC_NBUF = 4


def _sc_gather_sigmoid(x, encoder_weight):
    table_t = encoder_weight.T
    info = plsc.get_sparse_core_info()
    nc, ns = info.num_cores, info.num_subcores
    nw = nc * ns
    b_per_w = _BATCH // nw
    mesh = plsc.VectorSubcoreMesh(core_axis_name="c", subcore_axis_name="s")

    @functools.partial(
        pl.kernel,
        mesh=mesh,
        compiler_params=pltpu.CompilerParams(needs_layout_passes=False),
        out_type=jax.ShapeDtypeStruct((_BATCH, _LATENT_DIM), jnp.float32),
        scratch_types=[
            pltpu.VMEM((b_per_w,), jnp.int32),
        ] + [pltpu.VMEM((_LATENT_DIM, 128), jnp.float32)
             for _ in range(_SC_NBUF)] + [
            pltpu.VMEM((b_per_w, _LATENT_DIM), jnp.float32),
            pltpu.SemaphoreType.DMA((_SC_NBUF,)),
        ],
    )
    def body(x_hbm, table_hbm, out_hbm, idx_v, tb0, tb1, tb2, tb3, h_v, sems):
        tbufs = [tb0, tb1, tb2, tb3]
        wid = lax.axis_index("s") * nc + lax.axis_index("c")
        base = wid * b_per_w
        pltpu.sync_copy(x_hbm.at[pl.ds(base, b_per_w)], idx_v)

        lane = lax.iota(jnp.int32, 16)

        def x_scalar(i):
            chunk = idx_v[pl.ds((i // _LANES) * _LANES, _LANES)]
            return jnp.sum(jnp.where(lane == (i % _LANES), chunk, 0))

        xs = [x_scalar(i) for i in range(b_per_w)]

        def fire(i):
            c0 = (xs[i] // 128) * 128
            pltpu.make_async_copy(
                table_hbm.at[:, pl.ds(c0, 128)],
                tbufs[i % _SC_NBUF],
                sems.at[i % _SC_NBUF],
            ).start()

        for i in range(_SC_NBUF):
            fire(i)
        for i in range(b_per_w):
            slot = i % _SC_NBUF
            pltpu.make_async_copy(
                table_hbm.at[:, pl.ds(0, 128)],
                tbufs[slot],
                sems.at[slot],
            ).wait()
            c_vec = jnp.full((_LANES,), xs[i] - (xs[i] // 128) * 128,
                             dtype=jnp.int32)
            for g in range(_LATENT_DIM // _LANES):
                vals = plsc.load_gather(
                    tbufs[slot], [lane + g * _LANES, c_vec])
                h_v[i, pl.ds(g * _LANES, _LANES)] = vals
            if i + _SC_NBUF < b_per_w:
                fire(i + _SC_NBUF)
        pltpu.sync_copy(h_v, out_hbm.at[pl.ds(base, b_per_w)])

    return body(x, table_t)


_TILE_V = 6144


def _mm_body(h_ref, d_ref, o_ref):
    h = 1.0 / (1.0 + jnp.exp(-h_ref[...]))
    acc = jax.lax.dot_general(
        d_ref[...], h,
        (((0,), (1,)), ((), ())),
        preferred_element_type=jnp.float32,
    )
    o_ref[...] = 1.0 / (1.0 + jnp.exp(-acc))


def _tc_decode(h, decoder_weight):
    ot = pl.pallas_call(
        _mm_body,
        grid=(pl.cdiv(_INPUT_DIM, _TILE_V),),
        in_specs=[
            pl.BlockSpec((_BATCH, _LATENT_DIM), lambda i: (0, 0)),
            pl.BlockSpec((_LATENT_DIM, _TILE_V), lambda i: (0, i)),
        ],
        out_specs=pl.BlockSpec((_TILE_V, _BATCH), lambda i: (i, 0)),
        out_shape=jax.ShapeDtypeStruct((_INPUT_DIM, _BATCH), jnp.float32),
    )(h, decoder_weight)
    return ot.T


def kernel(x, encoder_weight, decoder_weight):
    h = _sc_gather_sigmoid(x.astype(jnp.int32), encoder_weight)
    return _tc_decode(h, decoder_weight)


# speedup vs baseline: 3.3169x; 1.0157x over previous
import functools

import jax
import jax.numpy as jnp
from jax import lax
from jax.experimental import pallas as pl
from jax.experimental.pallas import tpu as pltpu
from jax.experimental.pallas import tpu_sc as plsc

_INPUT_DIM = 100000
_LATENT_DIM = 64
_BATCH = 1024

_LANES = 16


_SC_NBUF = 8


def _sc_gather_sigmoid(x, encoder_weight):
    table_t = encoder_weight.T
    info = plsc.get_sparse_core_info()
    nc, ns = info.num_cores, info.num_subcores
    nw = nc * ns
    b_per_w = _BATCH // nw
    mesh = plsc.VectorSubcoreMesh(core_axis_name="c", subcore_axis_name="s")

    @functools.partial(
        pl.kernel,
        mesh=mesh,
        compiler_params=pltpu.CompilerParams(needs_layout_passes=False),
        out_type=jax.ShapeDtypeStruct((_BATCH, _LATENT_DIM), jnp.float32),
        scratch_types=[
            pltpu.VMEM((b_per_w,), jnp.int32),
        ] + [pltpu.VMEM((_LATENT_DIM, 128), jnp.float32)
             for _ in range(_SC_NBUF)] + [
            pltpu.VMEM((b_per_w, _LATENT_DIM), jnp.float32),
            pltpu.SemaphoreType.DMA((_SC_NBUF,)),
        ],
    )
    def body(x_hbm, table_hbm, out_hbm, idx_v,
             tb0, tb1, tb2, tb3, tb4, tb5, tb6, tb7, h_v, sems):
        tbufs = [tb0, tb1, tb2, tb3, tb4, tb5, tb6, tb7]
        wid = lax.axis_index("s") * nc + lax.axis_index("c")
        base = wid * b_per_w
        pltpu.sync_copy(x_hbm.at[pl.ds(base, b_per_w)], idx_v)

        lane = lax.iota(jnp.int32, 16)

        def x_scalar(i):
            chunk = idx_v[pl.ds((i // _LANES) * _LANES, _LANES)]
            return jnp.sum(jnp.where(lane == (i % _LANES), chunk, 0))

        xs = [x_scalar(i) for i in range(b_per_w)]

        def fire(i):
            c0 = (xs[i] // 128) * 128
            pltpu.make_async_copy(
                table_hbm.at[:, pl.ds(c0, 128)],
                tbufs[i % _SC_NBUF],
                sems.at[i % _SC_NBUF],
            ).start()

        for i in range(_SC_NBUF):
            fire(i)
        for i in range(b_per_w):
            slot = i % _SC_NBUF
            pltpu.make_async_copy(
                table_hbm.at[:, pl.ds(0, 128)],
                tbufs[slot],
                sems.at[slot],
            ).wait()
            c_vec = jnp.full((_LANES,), xs[i] - (xs[i] // 128) * 128,
                             dtype=jnp.int32)
            for g in range(_LATENT_DIM // _LANES):
                vals = plsc.load_gather(
                    tbufs[slot], [lane + g * _LANES, c_vec])
                h_v[i, pl.ds(g * _LANES, _LANES)] = vals
            if i + _SC_NBUF < b_per_w:
                fire(i + _SC_NBUF)
        pltpu.sync_copy(h_v, out_hbm.at[pl.ds(base, b_per_w)])

    return body(x, table_t)


_TILE_V = 6144


def _mm_body(h_ref, d_ref, o_ref):
    h = 1.0 / (1.0 + jnp.exp(-h_ref[...]))
    acc = jax.lax.dot_general(
        d_ref[...], h,
        (((0,), (1,)), ((), ())),
        preferred_element_type=jnp.float32,
    )
    o_ref[...] = 1.0 / (1.0 + jnp.exp(-acc))


def _tc_decode(h, decoder_weight):
    ot = pl.pallas_call(
        _mm_body,
        grid=(pl.cdiv(_INPUT_DIM, _TILE_V),),
        in_specs=[
            pl.BlockSpec((_BATCH, _LATENT_DIM), lambda i: (0, 0)),
            pl.BlockSpec((_LATENT_DIM, _TILE_V), lambda i: (0, i)),
        ],
        out_specs=pl.BlockSpec((_TILE_V, _BATCH), lambda i: (i, 0)),
        out_shape=jax.ShapeDtypeStruct((_INPUT_DIM, _BATCH), jnp.float32),
    )(h, decoder_weight)
    return ot.T


def kernel(x, encoder_weight, decoder_weight):
    h = _sc_gather_sigmoid(x.astype(jnp.int32), encoder_weight)
    return _tc_decode(h, decoder_weight)
